# baseline (speedup 1.0000x reference)
import functools

import jax
import jax.numpy as jnp
from jax import lax
from jax.experimental import pallas as pl
from jax.experimental.pallas import tpu as pltpu
from jax.experimental.pallas import tpu_sc as plsc

NC = 2
NS = 16
NW = NC * NS
LANES = 16


def _make_pool_kernel(B, L, V, D):
    assert B % NW == 0
    b_per_w = B // NW
    chunks = []
    off = 0
    while off < L:
        n = min(128, L - off)
        chunks.append((off, n))
        off += n
    n_j = D // LANES
    inv_l = 1.0 / float(L)

    mesh = plsc.VectorSubcoreMesh(
        core_axis_name="c", subcore_axis_name="s", num_cores=NC,
        num_subcores=NS)

    @functools.partial(
        pl.kernel,
        mesh=mesh,
        compiler_params=pltpu.CompilerParams(use_tc_tiling_on_sc=False),
        out_type=jax.ShapeDtypeStruct((B, D), jnp.float32),
        scratch_types=[
            pltpu.VMEM((b_per_w, L), jnp.int32),
            pltpu.VMEM((L, D), jnp.float32),
            pltpu.VMEM((L, D), jnp.float32),
            pltpu.VMEM((b_per_w, D), jnp.float32),
            pltpu.SemaphoreType.DMA,
            pltpu.SemaphoreType.DMA,
        ],
    )
    def pool(x_hbm, table_hbm, out_hbm, idx_v, buf0, buf1, pooled_v,
             sem0, sem1):
        wid = lax.axis_index("s") * NC + lax.axis_index("c")
        base = wid * b_per_w

        pltpu.sync_copy(x_hbm.at[pl.ds(base, b_per_w)], idx_v)

        def fire(r, buf, sem):
            for (o, n) in chunks:
                pltpu.async_copy(
                    table_hbm.at[idx_v.at[r, pl.ds(o, n)]],
                    buf.at[pl.ds(o, n)], sem)

        def drain(r, buf, sem):
            for (o, n) in chunks:
                pltpu.make_async_copy(
                    table_hbm.at[idx_v.at[r, pl.ds(o, n)]],
                    buf.at[pl.ds(o, n)], sem).wait()

        def accum(r, buf):
            def body(s, accs):
                return tuple(
                    a + buf[s, pl.ds(j * LANES, LANES)]
                    for j, a in enumerate(accs))
            accs = lax.fori_loop(
                0, L, body,
                tuple(jnp.zeros((LANES,), jnp.float32) for _ in range(n_j)))
            for j in range(n_j):
                pooled_v[r, pl.ds(j * LANES, LANES)] = accs[j] * inv_l

        fire(0, buf0, sem0)

        def body2(i, _):
            r = i * 2
            drain(r, buf0, sem0)
            fire(r + 1, buf1, sem1)
            accum(r, buf0)
            drain(r + 1, buf1, sem1)

            @pl.when(r + 2 < b_per_w)
            def _():
                fire(r + 2, buf0, sem0)

            accum(r + 1, buf1)
            return 0

        lax.fori_loop(0, b_per_w // 2, body2, 0)

        pltpu.sync_copy(pooled_v, out_hbm.at[pl.ds(base, b_per_w)])

    return pool


def _mm_body(p_ref, w_ref, b_ref, o_ref):
    o_ref[...] = lax.dot_general(
        p_ref[...], w_ref[...],
        dimension_numbers=(((1,), (1,)), ((), ())),
        preferred_element_type=jnp.float32) + b_ref[...]


def kernel(x, table, W, b):
    B, L = x.shape
    V, D = table.shape
    C = W.shape[0]

    pooled = _make_pool_kernel(B, L, V, D)(x.astype(jnp.int32), table)

    logit = pl.pallas_call(
        _mm_body,
        out_shape=jax.ShapeDtypeStruct((B, C), jnp.float32),
    )(pooled, W, b.reshape(1, C))
    return logit
